# baseline (speedup 1.0000x reference)
import functools

import jax
import jax.numpy as jnp
from jax import lax
from jax.experimental import pallas as pl
from jax.experimental.pallas import tpu as pltpu
from jax.experimental.pallas import tpu_sc as plsc

N = 10000
NPAD = 10240
D = 128
NG = 64
NC = 10
NW = 32
K = 64
CH = 168
IB = 8
NB = CH // IB
EPAD = NW * CH * K
N_EDGES_FULL = 330000
ROWS = NPAD // 16
RB = 1024


def _deg_body(dst_hbm, out_hbm, dst_b, obuf, zbuf, accd):
    cid = lax.axis_index("c")
    sid = lax.axis_index("s")
    wid = cid * 16 + sid

    def fill(i, _):
        for j in range(D // 16):
            zbuf[i, pl.ds(j * 16, 16)] = jnp.zeros((16,), jnp.float32)
            obuf[i, pl.ds(j * 16, 16)] = jnp.ones((16,), jnp.float32)
        return 0

    lax.fori_loop(0, K, fill, 0)
    for t in range(ROWS // K):
        pltpu.sync_copy(zbuf, accd.at[pl.ds(sid * ROWS + t * K, K)])
    plsc.subcore_barrier()

    def blk(b, _):
        pltpu.sync_copy(dst_hbm.at[wid, pl.ds(b * IB, IB)], dst_b)

        def body(j, _):
            pltpu.sync_copy(obuf, accd.at[dst_b.at[j]], add=True)
            return 0

        lax.fori_loop(0, IB, body, 0)
        return 0

    lax.fori_loop(0, NB, blk, 0)
    plsc.subcore_barrier()
    pltpu.sync_copy(accd.at[pl.ds(sid * ROWS, ROWS)],
                    out_hbm.at[cid, pl.ds(sid * ROWS, ROWS)])


def _agg_body(xw_hbm, src_hbm, dst_hbm, out_hbm,
              src_b, dst_b, buf0, buf1, buf2, buf3, acc,
              sem0, sem1, sem2, sem3):
    cid = lax.axis_index("c")
    sid = lax.axis_index("s")
    wid = cid * 16 + sid

    def zrow(i, _):
        for j in range(D // 16):
            buf0[i, pl.ds(j * 16, 16)] = jnp.zeros((16,), jnp.float32)
        return 0

    lax.fori_loop(0, K, zrow, 0)
    for t in range(ROWS // K):
        pltpu.sync_copy(buf0, acc.at[pl.ds(sid * ROWS + t * K, K)])
    plsc.subcore_barrier()

    def blk(b, _):
        pltpu.sync_copy(src_hbm.at[wid, pl.ds(b * IB, IB)], src_b)
        pltpu.sync_copy(dst_hbm.at[wid, pl.ds(b * IB, IB)], dst_b)

        def body(i, _):
            j0 = i * 4
            c0 = pltpu.async_copy(xw_hbm.at[src_b.at[j0]], buf0, sem0)
            c1 = pltpu.async_copy(xw_hbm.at[src_b.at[j0 + 1]], buf1, sem1)
            c2 = pltpu.async_copy(xw_hbm.at[src_b.at[j0 + 2]], buf2, sem2)
            c3 = pltpu.async_copy(xw_hbm.at[src_b.at[j0 + 3]], buf3, sem3)
            c0.wait()
            pltpu.sync_copy(buf0, acc.at[dst_b.at[j0]], add=True)
            c1.wait()
            pltpu.sync_copy(buf1, acc.at[dst_b.at[j0 + 1]], add=True)
            c2.wait()
            pltpu.sync_copy(buf2, acc.at[dst_b.at[j0 + 2]], add=True)
            c3.wait()
            pltpu.sync_copy(buf3, acc.at[dst_b.at[j0 + 3]], add=True)
            return 0

        lax.fori_loop(0, IB // 4, body, 0)
        return 0

    lax.fori_loop(0, NB, blk, 0)
    plsc.subcore_barrier()
    pltpu.sync_copy(acc.at[pl.ds(sid * ROWS, ROWS)],
                    out_hbm.at[cid, pl.ds(sid * ROWS, ROWS)])


@functools.cache
def _sc_kernels():
    mesh = plsc.VectorSubcoreMesh(core_axis_name="c", subcore_axis_name="s")
    deg_k = pl.kernel(
        _deg_body,
        out_type=jax.ShapeDtypeStruct((2, NPAD, D), jnp.float32),
        mesh=mesh,
        scratch_types=[
            pltpu.VMEM((IB, K), jnp.int32),
            pltpu.VMEM((K, D), jnp.float32),
            pltpu.VMEM((K, D), jnp.float32),
            pltpu.VMEM_SHARED((NPAD, D), jnp.float32),
        ],
    )
    agg_k = pl.kernel(
        _agg_body,
        out_type=jax.ShapeDtypeStruct((2, NPAD, D), jnp.float32),
        mesh=mesh,
        scratch_types=[
            pltpu.VMEM((IB, K), jnp.int32),
            pltpu.VMEM((IB, K), jnp.int32),
            pltpu.VMEM((K, D), jnp.float32),
            pltpu.VMEM((K, D), jnp.float32),
            pltpu.VMEM((K, D), jnp.float32),
            pltpu.VMEM((K, D), jnp.float32),
            pltpu.VMEM_SHARED((NPAD, D), jnp.float32),
            pltpu.SemaphoreType.DMA,
            pltpu.SemaphoreType.DMA,
            pltpu.SemaphoreType.DMA,
            pltpu.SemaphoreType.DMA,
        ],
    )
    return deg_k, agg_k


def _mm1_body(x_ref, w_ref, d0_ref, d1_ref, xw_ref, dinv_ref):
    deg = d0_ref[:, 0:1] + d1_ref[:, 0:1]
    dinv = lax.rsqrt(jnp.maximum(deg, 1.0))
    dinv_ref[...] = dinv
    xw_ref[...] = jnp.dot(x_ref[...], w_ref[...],
                          preferred_element_type=jnp.float32) * dinv


_mm1 = pl.pallas_call(
    _mm1_body,
    grid=(NPAD // RB,),
    in_specs=[
        pl.BlockSpec((RB, D), lambda i: (i, 0)),
        pl.BlockSpec((D, D), lambda i: (0, 0)),
        pl.BlockSpec((RB, D), lambda i: (i, 0)),
        pl.BlockSpec((RB, D), lambda i: (i, 0)),
    ],
    out_specs=[
        pl.BlockSpec((RB, D), lambda i: (i, 0)),
        pl.BlockSpec((RB, 1), lambda i: (i, 0)),
    ],
    out_shape=[
        jax.ShapeDtypeStruct((NPAD, D), jnp.float32),
        jax.ShapeDtypeStruct((NPAD, 1), jnp.float32),
    ],
)


def _mid_body(p0_ref, p1_ref, dinv_ref, b_ref, w_ref, xw_ref):
    dinv = dinv_ref[...]
    h = jnp.maximum((p0_ref[...] + p1_ref[...]) * dinv + b_ref[...], 0.0)
    xw_ref[...] = jnp.dot(h, w_ref[...],
                          preferred_element_type=jnp.float32) * dinv


_mid = pl.pallas_call(
    _mid_body,
    grid=(NPAD // RB,),
    in_specs=[
        pl.BlockSpec((RB, D), lambda i: (i, 0)),
        pl.BlockSpec((RB, D), lambda i: (i, 0)),
        pl.BlockSpec((RB, 1), lambda i: (i, 0)),
        pl.BlockSpec((1, D), lambda i: (0, 0)),
        pl.BlockSpec((D, D), lambda i: (0, 0)),
    ],
    out_specs=pl.BlockSpec((RB, D), lambda i: (i, 0)),
    out_shape=jax.ShapeDtypeStruct((NPAD, D), jnp.float32),
)


def _fin_body(p0_ref, p1_ref, dinv_ref, bt_ref, b3_ref, wl_ref, bl_ref,
              out_ref, sums, counts):
    i = pl.program_id(0)

    @pl.when(i == 0)
    def _init():
        sums[...] = jnp.zeros_like(sums)
        counts[...] = jnp.zeros_like(counts)

    h = (p0_ref[...] + p1_ref[...]) * dinv_ref[...]
    pt = (bt_ref[...] == lax.broadcasted_iota(jnp.int32, (NG, RB), 0)
          ).astype(jnp.float32)
    sums[...] += jnp.dot(pt, h, preferred_element_type=jnp.float32)
    counts[...] += jnp.sum(pt, axis=1, keepdims=True)

    @pl.when(i == pl.num_programs(0) - 1)
    def _fin():
        cnt = counts[...]
        pooled = (sums[...] / jnp.maximum(cnt, 1.0)
                  + jnp.where(cnt > 0, b3_ref[...], 0.0))
        out_ref[...] = jnp.dot(pooled, wl_ref[...],
                               preferred_element_type=jnp.float32) + bl_ref[...]


_fin = pl.pallas_call(
    _fin_body,
    grid=(NPAD // RB,),
    in_specs=[
        pl.BlockSpec((RB, D), lambda i: (i, 0)),
        pl.BlockSpec((RB, D), lambda i: (i, 0)),
        pl.BlockSpec((RB, 1), lambda i: (i, 0)),
        pl.BlockSpec((1, RB), lambda i: (0, i)),
        pl.BlockSpec((1, D), lambda i: (0, 0)),
        pl.BlockSpec((D, NC), lambda i: (0, 0)),
        pl.BlockSpec((1, NC), lambda i: (0, 0)),
    ],
    out_specs=pl.BlockSpec((NG, NC), lambda i: (0, 0)),
    out_shape=jax.ShapeDtypeStruct((NG, NC), jnp.float32),
    scratch_shapes=[
        pltpu.VMEM((NG, D), jnp.float32),
        pltpu.VMEM((NG, 1), jnp.float32),
    ],
)


def kernel(x, edge_index, batch, W1, b1, W2, b2, W3, b3, Wl, bl):
    loop = jnp.arange(N, dtype=jnp.int32)
    pad = jnp.full((EPAD - N_EDGES_FULL,), N, dtype=jnp.int32)
    src = jnp.concatenate([edge_index[0].astype(jnp.int32), loop, pad])
    dst = jnp.concatenate([edge_index[1].astype(jnp.int32), loop, pad])
    src, dst = lax.sort([src, dst], num_keys=1)
    src3 = src.reshape(NW, CH, K)
    dst3 = dst.reshape(NW, CH, K)

    x_pad = jnp.pad(x, ((0, NPAD - N), (0, 0)))
    bt = jnp.pad(batch.astype(jnp.int32), (0, NPAD - N),
                 constant_values=NG)[None, :]

    deg_k, agg_k = _sc_kernels()
    degp = deg_k(dst3)
    xw, dinv = _mm1(x_pad, W1, degp[0], degp[1])
    p = agg_k(xw, src3, dst3)
    xw = _mid(p[0], p[1], dinv, b1.reshape(1, D), W2)
    p = agg_k(xw, src3, dst3)
    xw = _mid(p[0], p[1], dinv, b2.reshape(1, D), W3)
    p = agg_k(xw, src3, dst3)
    return _fin(p[0], p[1], dinv, bt, b3.reshape(1, D),
                Wl, bl.reshape(1, NC))


# speedup vs baseline: 4.0648x; 4.0648x over previous
import functools

import jax
import jax.numpy as jnp
from jax import lax
from jax.experimental import pallas as pl
from jax.experimental.pallas import tpu as pltpu
from jax.experimental.pallas import tpu_sc as plsc

N = 10000
NPAD = 10240
D = 128
NG = 64
NC = 10
NW = 32
K = 64
CH = 168
IB = 8
NB = CH // IB
EPAD = NW * CH * K
N_EDGES_FULL = 330000
ROWS = NPAD // 16
RB = 1024


def _deg_body(dst_hbm, out_hbm, dst_b, obuf, zbuf, accd):
    cid = lax.axis_index("c")
    sid = lax.axis_index("s")
    wid = cid * 16 + sid

    def fill(i, _):
        for j in range(D // 16):
            zbuf[i, pl.ds(j * 16, 16)] = jnp.zeros((16,), jnp.float32)
            obuf[i, pl.ds(j * 16, 16)] = jnp.ones((16,), jnp.float32)
        return 0

    lax.fori_loop(0, K, fill, 0)
    for t in range(ROWS // K):
        pltpu.sync_copy(zbuf, accd.at[pl.ds(sid * ROWS + t * K, K)])
    plsc.subcore_barrier()

    def blk(b, _):
        pltpu.sync_copy(dst_hbm.at[wid, pl.ds(b * IB, IB)], dst_b)

        def body(j, _):
            pltpu.sync_copy(obuf, accd.at[dst_b.at[j]], add=True)
            return 0

        lax.fori_loop(0, IB, body, 0)
        return 0

    lax.fori_loop(0, NB, blk, 0)
    plsc.subcore_barrier()
    pltpu.sync_copy(accd.at[pl.ds(sid * ROWS, ROWS)],
                    out_hbm.at[cid, pl.ds(sid * ROWS, ROWS)])


def _agg_body(xw_hbm, src_hbm, dst_hbm, out_hbm,
              src_b, dst_b, buf0, buf1, buf2, buf3, acc,
              sem0, sem1, sem2, sem3):
    cid = lax.axis_index("c")
    sid = lax.axis_index("s")
    wid = cid * 16 + sid

    def zrow(i, _):
        for j in range(D // 16):
            buf0[i, pl.ds(j * 16, 16)] = jnp.zeros((16,), jnp.float32)
        return 0

    lax.fori_loop(0, K, zrow, 0)
    for t in range(ROWS // K):
        pltpu.sync_copy(buf0, acc.at[pl.ds(sid * ROWS + t * K, K)])
    plsc.subcore_barrier()

    def blk(b, _):
        pltpu.sync_copy(src_hbm.at[wid, pl.ds(b * IB, IB)], src_b)
        pltpu.sync_copy(dst_hbm.at[wid, pl.ds(b * IB, IB)], dst_b)

        def body(i, _):
            j0 = i * 4
            c0 = pltpu.async_copy(xw_hbm.at[src_b.at[j0]], buf0, sem0)
            c1 = pltpu.async_copy(xw_hbm.at[src_b.at[j0 + 1]], buf1, sem1)
            c2 = pltpu.async_copy(xw_hbm.at[src_b.at[j0 + 2]], buf2, sem2)
            c3 = pltpu.async_copy(xw_hbm.at[src_b.at[j0 + 3]], buf3, sem3)
            c0.wait()
            pltpu.sync_copy(buf0, acc.at[dst_b.at[j0]], add=True)
            c1.wait()
            pltpu.sync_copy(buf1, acc.at[dst_b.at[j0 + 1]], add=True)
            c2.wait()
            pltpu.sync_copy(buf2, acc.at[dst_b.at[j0 + 2]], add=True)
            c3.wait()
            pltpu.sync_copy(buf3, acc.at[dst_b.at[j0 + 3]], add=True)
            return 0

        lax.fori_loop(0, IB // 4, body, 0)
        return 0

    lax.fori_loop(0, NB, blk, 0)
    plsc.subcore_barrier()
    pltpu.sync_copy(acc.at[pl.ds(sid * ROWS, ROWS)],
                    out_hbm.at[cid, pl.ds(sid * ROWS, ROWS)])


@functools.cache
def _sc_kernels():
    mesh = plsc.VectorSubcoreMesh(core_axis_name="c", subcore_axis_name="s")
    deg_k = pl.kernel(
        _deg_body,
        out_type=jax.ShapeDtypeStruct((2, NPAD, D), jnp.float32),
        mesh=mesh,
        scratch_types=[
            pltpu.VMEM((IB, K), jnp.int32),
            pltpu.VMEM((K, D), jnp.float32),
            pltpu.VMEM((K, D), jnp.float32),
            pltpu.VMEM_SHARED((NPAD, D), jnp.float32),
        ],
    )
    agg_k = pl.kernel(
        _agg_body,
        out_type=jax.ShapeDtypeStruct((2, NPAD, D), jnp.float32),
        mesh=mesh,
        scratch_types=[
            pltpu.VMEM((IB, K), jnp.int32),
            pltpu.VMEM((IB, K), jnp.int32),
            pltpu.VMEM((K, D), jnp.float32),
            pltpu.VMEM((K, D), jnp.float32),
            pltpu.VMEM((K, D), jnp.float32),
            pltpu.VMEM((K, D), jnp.float32),
            pltpu.VMEM_SHARED((NPAD, D), jnp.float32),
            pltpu.SemaphoreType.DMA,
            pltpu.SemaphoreType.DMA,
            pltpu.SemaphoreType.DMA,
            pltpu.SemaphoreType.DMA,
        ],
    )
    return deg_k, agg_k


def _mm1_body(x_ref, w_ref, d0_ref, d1_ref, xw_ref, dinv_ref):
    deg = d0_ref[:, 0:1] + d1_ref[:, 0:1]
    dinv = lax.rsqrt(jnp.maximum(deg, 1.0))
    dinv_ref[...] = dinv
    xw_ref[...] = jnp.dot(x_ref[...], w_ref[...],
                          preferred_element_type=jnp.float32) * dinv


_mm1 = pl.pallas_call(
    _mm1_body,
    grid=(NPAD // RB,),
    in_specs=[
        pl.BlockSpec((RB, D), lambda i: (i, 0)),
        pl.BlockSpec((D, D), lambda i: (0, 0)),
        pl.BlockSpec((RB, D), lambda i: (i, 0)),
        pl.BlockSpec((RB, D), lambda i: (i, 0)),
    ],
    out_specs=[
        pl.BlockSpec((RB, D), lambda i: (i, 0)),
        pl.BlockSpec((RB, 1), lambda i: (i, 0)),
    ],
    out_shape=[
        jax.ShapeDtypeStruct((NPAD, D), jnp.float32),
        jax.ShapeDtypeStruct((NPAD, 1), jnp.float32),
    ],
)


def _mid_body(p0_ref, p1_ref, dinv_ref, b_ref, w_ref, xw_ref):
    dinv = dinv_ref[...]
    h = jnp.maximum((p0_ref[...] + p1_ref[...]) * dinv + b_ref[...], 0.0)
    xw_ref[...] = jnp.dot(h, w_ref[...],
                          preferred_element_type=jnp.float32) * dinv


_mid = pl.pallas_call(
    _mid_body,
    grid=(NPAD // RB,),
    in_specs=[
        pl.BlockSpec((RB, D), lambda i: (i, 0)),
        pl.BlockSpec((RB, D), lambda i: (i, 0)),
        pl.BlockSpec((RB, 1), lambda i: (i, 0)),
        pl.BlockSpec((1, D), lambda i: (0, 0)),
        pl.BlockSpec((D, D), lambda i: (0, 0)),
    ],
    out_specs=pl.BlockSpec((RB, D), lambda i: (i, 0)),
    out_shape=jax.ShapeDtypeStruct((NPAD, D), jnp.float32),
)


def _fin_body(p0_ref, p1_ref, dinv_ref, bt_ref, b3_ref, wl_ref, bl_ref,
              out_ref, sums, counts):
    i = pl.program_id(0)

    @pl.when(i == 0)
    def _init():
        sums[...] = jnp.zeros_like(sums)
        counts[...] = jnp.zeros_like(counts)

    h = (p0_ref[...] + p1_ref[...]) * dinv_ref[...]
    pt = (bt_ref[...] == lax.broadcasted_iota(jnp.int32, (NG, RB), 0)
          ).astype(jnp.float32)
    sums[...] += jnp.dot(pt, h, preferred_element_type=jnp.float32)
    counts[...] += jnp.sum(pt, axis=1, keepdims=True)

    @pl.when(i == pl.num_programs(0) - 1)
    def _fin():
        cnt = counts[...]
        pooled = (sums[...] / jnp.maximum(cnt, 1.0)
                  + jnp.where(cnt > 0, b3_ref[...], 0.0))
        out_ref[...] = jnp.dot(pooled, wl_ref[...],
                               preferred_element_type=jnp.float32) + bl_ref[...]


_fin = pl.pallas_call(
    _fin_body,
    grid=(NPAD // RB,),
    in_specs=[
        pl.BlockSpec((RB, D), lambda i: (i, 0)),
        pl.BlockSpec((RB, D), lambda i: (i, 0)),
        pl.BlockSpec((RB, 1), lambda i: (i, 0)),
        pl.BlockSpec((1, RB), lambda i: (0, i)),
        pl.BlockSpec((1, D), lambda i: (0, 0)),
        pl.BlockSpec((D, NC), lambda i: (0, 0)),
        pl.BlockSpec((1, NC), lambda i: (0, 0)),
    ],
    out_specs=pl.BlockSpec((NG, NC), lambda i: (0, 0)),
    out_shape=jax.ShapeDtypeStruct((NG, NC), jnp.float32),
    scratch_shapes=[
        pltpu.VMEM((NG, D), jnp.float32),
        pltpu.VMEM((NG, 1), jnp.float32),
    ],
)


def kernel(x, edge_index, batch, W1, b1, W2, b2, W3, b3, Wl, bl):
    loop = jnp.arange(N, dtype=jnp.int32)
    pad = N + jnp.arange(EPAD - N_EDGES_FULL, dtype=jnp.int32) % (NPAD - N)
    src = jnp.concatenate([edge_index[0].astype(jnp.int32), loop, pad])
    dst = jnp.concatenate([edge_index[1].astype(jnp.int32), loop, pad])
    src3 = src.reshape(NW, CH, K)
    dst3 = dst.reshape(NW, CH, K)

    x_pad = jnp.pad(x, ((0, NPAD - N), (0, 0)))
    bt = jnp.pad(batch.astype(jnp.int32), (0, NPAD - N),
                 constant_values=NG)[None, :]

    deg_k, agg_k = _sc_kernels()
    degp = deg_k(dst3)
    xw, dinv = _mm1(x_pad, W1, degp[0], degp[1])
    p = agg_k(xw, src3, dst3)
    xw = _mid(p[0], p[1], dinv, b1.reshape(1, D), W2)
    p = agg_k(xw, src3, dst3)
    xw = _mid(p[0], p[1], dinv, b2.reshape(1, D), W3)
    p = agg_k(xw, src3, dst3)
    return _fin(p[0], p[1], dinv, bt, b3.reshape(1, D),
                Wl, bl.reshape(1, NC))
